# baseline (speedup 1.0000x reference)
import jax
import jax.numpy as jnp
from jax.experimental import pallas as pl
from jax.experimental.pallas import tpu as pltpu

_HID = 2048
_NE = 64
_K = 8


def _router_block(x_ref, wt_ref, b_ref, idx_ref, wgt_ref, acc_ref):
    j = pl.program_id(1)
    part = jnp.dot(x_ref[...], wt_ref[...], preferred_element_type=jnp.float32)

    @pl.when(j == 0)
    def _first():
        acc_ref[...] = part

    @pl.when(j == 1)
    def _last():
        logits = acc_ref[...] + part
        logits_t = logits.T
        scores = jax.nn.sigmoid(logits_t)
        biased = scores + b_ref[...]

        row = jax.lax.broadcasted_iota(jnp.int32, biased.shape, 0)
        cur = biased
        picked_i = []
        picked_w = []
        for _ in range(_K):
            m = jnp.max(cur, axis=0, keepdims=True)
            eq = cur == m
            idx = jnp.min(jnp.where(eq, row, _NE), axis=0, keepdims=True)
            onehot = row == idx
            w = jnp.sum(jnp.where(onehot, scores, 0.0), axis=0, keepdims=True)
            picked_i.append(idx)
            picked_w.append(w)
            cur = jnp.where(onehot, -jnp.inf, cur)

        idx_t = jnp.concatenate(picked_i, axis=0)
        wgt_t = jnp.concatenate(picked_w, axis=0)
        denom = jnp.sum(wgt_t, axis=0, keepdims=True) + 1e-20
        idx_ref[...] = idx_t.T
        wgt_ref[...] = (wgt_t / denom).T


@jax.jit
def kernel(hidden_states, weight, e_score_correction_bias):
    x = hidden_states.reshape(-1, _HID).astype(jnp.float32)
    t = x.shape[0]
    bt = 2048
    bh = _HID // 2
    wt = weight.astype(jnp.float32).T
    bias = e_score_correction_bias.astype(jnp.float32).reshape(_NE, 1)

    grid = (t // bt, 2)
    out = pl.pallas_call(
        _router_block,
        grid=grid,
        in_specs=[
            pl.BlockSpec((bt, bh), lambda i, j: (i, j)),
            pl.BlockSpec((bh, _NE), lambda i, j: (j, 0)),
            pl.BlockSpec((_NE, 1), lambda i, j: (0, 0)),
        ],
        out_specs=[
            pl.BlockSpec((bt, _K), lambda i, j: (i, 0)),
            pl.BlockSpec((bt, _K), lambda i, j: (i, 0)),
        ],
        out_shape=[
            jax.ShapeDtypeStruct((t, _K), jnp.int32),
            jax.ShapeDtypeStruct((t, _K), jnp.float32),
        ],
        scratch_shapes=[pltpu.VMEM((bt, _NE), jnp.float32)],
    )(x, wt, bias)
    return out[0], out[1]


# speedup vs baseline: 1.2057x; 1.2057x over previous
import jax
import jax.numpy as jnp
from jax.experimental import pallas as pl

_HID = 2048
_NE = 64
_K = 8


def _router_block(x_ref, wt_ref, b_ref, idx_ref, wgt_ref):
    x = x_ref[...]
    wt = wt_ref[...]
    logits = jnp.dot(x, wt, preferred_element_type=jnp.float32)
    logits_t = logits.T
    scores = jax.nn.sigmoid(logits_t)
    biased = scores + b_ref[...]

    row = jax.lax.broadcasted_iota(jnp.int32, biased.shape, 0)
    cur = biased
    picked_i = []
    picked_w = []
    for _ in range(_K):
        m = jnp.max(cur, axis=0, keepdims=True)
        eq = cur == m
        idx = jnp.min(jnp.where(eq, row, _NE), axis=0, keepdims=True)
        onehot = row == idx
        w = jnp.sum(jnp.where(onehot, scores, 0.0), axis=0, keepdims=True)
        picked_i.append(idx)
        picked_w.append(w)
        cur = jnp.where(onehot, -jnp.inf, cur)

    idx_t = jnp.concatenate(picked_i, axis=0)
    wgt_t = jnp.concatenate(picked_w, axis=0)
    denom = jnp.sum(wgt_t, axis=0, keepdims=True) + 1e-20
    idx_ref[...] = idx_t.T
    wgt_ref[...] = (wgt_t / denom).T


@jax.jit
def kernel(hidden_states, weight, e_score_correction_bias):
    x = hidden_states.reshape(-1, _HID).astype(jnp.float32)
    t = x.shape[0]
    bt = 2048
    wt = weight.astype(jnp.float32).T
    bias = e_score_correction_bias.astype(jnp.float32).reshape(_NE, 1)

    grid = (t // bt,)
    out = pl.pallas_call(
        _router_block,
        grid=grid,
        in_specs=[
            pl.BlockSpec((bt, _HID), lambda i: (i, 0)),
            pl.BlockSpec((_HID, _NE), lambda i: (0, 0)),
            pl.BlockSpec((_NE, 1), lambda i: (0, 0)),
        ],
        out_specs=[
            pl.BlockSpec((bt, _K), lambda i: (i, 0)),
            pl.BlockSpec((bt, _K), lambda i: (i, 0)),
        ],
        out_shape=[
            jax.ShapeDtypeStruct((t, _K), jnp.int32),
            jax.ShapeDtypeStruct((t, _K), jnp.float32),
        ],
    )(x, wt, bias)
    return out[0], out[1]
